# baseline (speedup 1.0000x reference)
import dataclasses
import functools

import jax
import jax.numpy as jnp
from jax import lax
from jax.experimental import pallas as pl
from jax.experimental.pallas import tpu as pltpu
from jax.experimental.pallas import tpu_sc as plsc

VOCAB = 50368
HIDDEN = 768
EPS = 1e-05
BATCH = 4
SEQ = 8192

NUM_TOKENS = BATCH * SEQ
NC = 2
NS = 16
NW = NC * NS
B_PER_W = NUM_TOKENS // NW
CHUNK = 64
N_CHUNKS = B_PER_W // CHUNK
LANES = 16
NVEC = HIDDEN // LANES


def _layernorm_chunk(buf, wv):
    @pl.loop(0, CHUNK)
    def _(r):
        acc_s = jnp.zeros((LANES,), jnp.float32)
        acc_q = jnp.zeros((LANES,), jnp.float32)
        for v in range(NVEC):
            x = buf[r, pl.ds(v * LANES, LANES)]
            acc_s = acc_s + x
            acc_q = acc_q + x * x
        s = jnp.sum(acc_s)
        q = jnp.sum(acc_q)
        mean = s * (1.0 / HIDDEN)
        var = q * (1.0 / HIDDEN) - mean * mean + EPS
        seed = jnp.int32(0x5F3759DF) - lax.shift_right_arithmetic(
            lax.bitcast_convert_type(var, jnp.int32), 1)
        y = lax.bitcast_convert_type(seed, jnp.float32)
        half_var = 0.5 * var
        for _ in range(3):
            y = y * (1.5 - half_var * y * y)
        for v in range(NVEC):
            sl = pl.ds(v * LANES, LANES)
            x = buf[r, sl]
            buf[r, sl] = (x - mean) * y * wv[sl]


def _sc_gather_layernorm(table, idx_flat, w):
    mesh = plsc.VectorSubcoreMesh(core_axis_name="c", subcore_axis_name="s")
    cp = pltpu.CompilerParams()
    if "needs_layout_passes" in pltpu.CompilerParams.__dataclass_fields__:
        cp = dataclasses.replace(cp, needs_layout_passes=False)

    @functools.partial(
        pl.kernel,
        out_type=jax.ShapeDtypeStruct((NUM_TOKENS, HIDDEN), jnp.float32),
        mesh=mesh,
        compiler_params=cp,
        scratch_types=[
            pltpu.VMEM((B_PER_W,), jnp.int32),
            pltpu.VMEM((HIDDEN,), jnp.float32),
            pltpu.VMEM((CHUNK, HIDDEN), jnp.float32),
            pltpu.VMEM((CHUNK, HIDDEN), jnp.float32),
            pltpu.SemaphoreType.DMA,
            pltpu.SemaphoreType.DMA,
            pltpu.SemaphoreType.DMA,
            pltpu.SemaphoreType.DMA,
        ],
    )
    def fused_kernel(table_hbm, idx_hbm, w_hbm, out_hbm,
                     idx_v, wv, buf_a, buf_b, gsem_a, gsem_b, wsem_a, wsem_b):
        wid = lax.axis_index("s") * NC + lax.axis_index("c")
        base = wid * B_PER_W
        pltpu.sync_copy(idx_hbm.at[pl.ds(base, B_PER_W)], idx_v)
        pltpu.sync_copy(w_hbm, wv)

        bufs = (buf_a, buf_b)
        gsems = (gsem_a, gsem_b)
        wsems = (wsem_a, wsem_b)

        def start_gather(k, j):
            pltpu.async_copy(
                table_hbm.at[idx_v.at[pl.ds(k * CHUNK, CHUNK)]],
                bufs[j], gsems[j])

        def wait_gather(k, j):
            pltpu.make_async_copy(
                table_hbm.at[idx_v.at[pl.ds(k * CHUNK, CHUNK)]],
                bufs[j], gsems[j]).wait()

        def start_wb(k, j):
            pltpu.async_copy(
                bufs[j], out_hbm.at[pl.ds(base + k * CHUNK, CHUNK)], wsems[j])

        def wait_wb(k, j):
            pltpu.make_async_copy(
                bufs[j], out_hbm.at[pl.ds(base + k * CHUNK, CHUNK)],
                wsems[j]).wait()

        start_gather(0, 0)

        @pl.loop(0, N_CHUNKS, step=2)
        def _(c):
            wait_gather(c, 0)

            @pl.when(c > 0)
            def _():
                wait_wb(c - 1, 1)

            start_gather(c + 1, 1)
            _layernorm_chunk(buf_a, wv)
            start_wb(c, 0)

            wait_gather(c + 1, 1)
            _layernorm_chunk(buf_b, wv)
            wait_wb(c, 0)

            @pl.when(c + 2 < N_CHUNKS)
            def _():
                start_gather(c + 2, 0)

            start_wb(c + 1, 1)

        wait_wb(N_CHUNKS - 1, 1)

    return fused_kernel(table, idx_flat, w)


def kernel(input_ids, tok_embeddings, norm_weight):
    idx_flat = input_ids.reshape(NUM_TOKENS)
    normed = _sc_gather_layernorm(tok_embeddings, idx_flat, norm_weight)
    return normed.reshape(BATCH, SEQ, HIDDEN)


# speedup vs baseline: 2.0807x; 2.0807x over previous
import dataclasses
import functools

import jax
import jax.numpy as jnp
from jax import lax
from jax.experimental import pallas as pl
from jax.experimental.pallas import tpu as pltpu
from jax.experimental.pallas import tpu_sc as plsc

VOCAB = 50368
HIDDEN = 768
EPS = 1e-05
BATCH = 4
SEQ = 8192

NUM_TOKENS = BATCH * SEQ
NC = 2
NS = 16
NW = NC * NS
B_PER_W = NUM_TOKENS // NW
CHUNK = 64
N_CHUNKS = B_PER_W // CHUNK
LANES = 16
NVEC = HIDDEN // LANES


RI = 4


def _layernorm_chunk(buf, wv):
    @pl.loop(0, CHUNK, step=RI)
    def _(r0):
        acc_s = [jnp.zeros((LANES,), jnp.float32) for _ in range(RI)]
        acc_q = [jnp.zeros((LANES,), jnp.float32) for _ in range(RI)]
        for v in range(NVEC):
            sl = pl.ds(v * LANES, LANES)
            for i in range(RI):
                x = buf[r0 + i, sl]
                acc_s[i] = acc_s[i] + x
                acc_q[i] = acc_q[i] + x * x
        s = [jnp.sum(a) for a in acc_s]
        q = [jnp.sum(a) for a in acc_q]
        mean = [si * (1.0 / HIDDEN) for si in s]
        var = [qi * (1.0 / HIDDEN) - mi * mi + EPS for qi, mi in zip(q, mean)]
        y = [lax.bitcast_convert_type(
                jnp.int32(0x5F3759DF) - lax.shift_right_arithmetic(
                    lax.bitcast_convert_type(vi, jnp.int32), 1),
                jnp.float32) for vi in var]
        half_var = [0.5 * vi for vi in var]
        for _ in range(3):
            y = [yi * (1.5 - hi * yi * yi) for yi, hi in zip(y, half_var)]
        for v in range(NVEC):
            sl = pl.ds(v * LANES, LANES)
            wvv = wv[sl]
            for i in range(RI):
                x = buf[r0 + i, sl]
                buf[r0 + i, sl] = (x - mean[i]) * y[i] * wvv


def _sc_gather_layernorm(table, idx_flat, w):
    mesh = plsc.VectorSubcoreMesh(core_axis_name="c", subcore_axis_name="s")
    cp = pltpu.CompilerParams()
    if "needs_layout_passes" in pltpu.CompilerParams.__dataclass_fields__:
        cp = dataclasses.replace(cp, needs_layout_passes=False)

    @functools.partial(
        pl.kernel,
        out_type=jax.ShapeDtypeStruct((NUM_TOKENS, HIDDEN), jnp.float32),
        mesh=mesh,
        compiler_params=cp,
        scratch_types=[
            pltpu.VMEM((B_PER_W,), jnp.int32),
            pltpu.VMEM((HIDDEN,), jnp.float32),
            pltpu.VMEM((CHUNK, HIDDEN), jnp.float32),
            pltpu.VMEM((CHUNK, HIDDEN), jnp.float32),
            pltpu.SemaphoreType.DMA,
            pltpu.SemaphoreType.DMA,
            pltpu.SemaphoreType.DMA,
            pltpu.SemaphoreType.DMA,
        ],
    )
    def fused_kernel(table_hbm, idx_hbm, w_hbm, out_hbm,
                     idx_v, wv, buf_a, buf_b, gsem_a, gsem_b, wsem_a, wsem_b):
        wid = lax.axis_index("s") * NC + lax.axis_index("c")
        base = wid * B_PER_W
        pltpu.sync_copy(idx_hbm.at[pl.ds(base, B_PER_W)], idx_v)
        pltpu.sync_copy(w_hbm, wv)

        bufs = (buf_a, buf_b)
        gsems = (gsem_a, gsem_b)
        wsems = (wsem_a, wsem_b)

        def start_gather(k, j):
            pltpu.async_copy(
                table_hbm.at[idx_v.at[pl.ds(k * CHUNK, CHUNK)]],
                bufs[j], gsems[j])

        def wait_gather(k, j):
            pltpu.make_async_copy(
                table_hbm.at[idx_v.at[pl.ds(k * CHUNK, CHUNK)]],
                bufs[j], gsems[j]).wait()

        def start_wb(k, j):
            pltpu.async_copy(
                bufs[j], out_hbm.at[pl.ds(base + k * CHUNK, CHUNK)], wsems[j])

        def wait_wb(k, j):
            pltpu.make_async_copy(
                bufs[j], out_hbm.at[pl.ds(base + k * CHUNK, CHUNK)],
                wsems[j]).wait()

        start_gather(0, 0)

        @pl.loop(0, N_CHUNKS, step=2)
        def _(c):
            wait_gather(c, 0)

            @pl.when(c > 0)
            def _():
                wait_wb(c - 1, 1)

            start_gather(c + 1, 1)
            _layernorm_chunk(buf_a, wv)
            start_wb(c, 0)

            wait_gather(c + 1, 1)
            _layernorm_chunk(buf_b, wv)
            wait_wb(c, 0)

            @pl.when(c + 2 < N_CHUNKS)
            def _():
                start_gather(c + 2, 0)

            start_wb(c + 1, 1)

        wait_wb(N_CHUNKS - 1, 1)

    return fused_kernel(table, idx_flat, w)


def kernel(input_ids, tok_embeddings, norm_weight):
    idx_flat = input_ids.reshape(NUM_TOKENS)
    normed = _sc_gather_layernorm(tok_embeddings, idx_flat, norm_weight)
    return normed.reshape(BATCH, SEQ, HIDDEN)
